# baseline (speedup 1.0000x reference)
import dataclasses
from math import pi as PI, sqrt

import jax
import jax.numpy as jnp
from jax import lax
from jax.experimental import pallas as pl
from jax.experimental.pallas import tpu as pltpu
from jax.experimental.pallas import tpu_sc as plsc

N = 1048576
NW = 16
CHUNK = N // NW
BLK = 8192
NBLK = CHUNK // BLK
NROW = BLK // 128
_PREF = 1.0 / sqrt(2.0 * PI)

_i32 = jnp.int32
_u32 = jnp.uint32
_f32 = jnp.float32


def _cparams():
    cp = pltpu.CompilerParams()
    if "needs_layout_passes" in pltpu.CompilerParams.__dataclass_fields__:
        cp = dataclasses.replace(cp, needs_layout_passes=False)
    return cp


def _mesh():
    return plsc.VectorSubcoreMesh(
        core_axis_name="c", subcore_axis_name="s", num_cores=1)


def _to_key(fbits):
    top = fbits >> _u32(31)
    return jnp.where(top == _u32(1), ~fbits, fbits | _u32(0x80000000))


def _from_key(k):
    top = k >> _u32(31)
    bits = jnp.where(top == _u32(1), k & _u32(0x7FFFFFFF), ~k)
    return lax.bitcast_convert_type(bits, _f32)


def _pos_transform(r):
    w = lax.shift_right_logical(r, 16)
    rem = r & 0xFFFF
    l = lax.shift_right_logical(rem, 12)
    v = rem & 0xFFF
    return (w << 16) | (v << 4) | l


def _radix_body(shift, first, final):
    def body(in_hbm, out_hbm, inbuf_f, inbuf_u, posbuf, cnt, off,
             tbuf, tall, gbuf, pbuf, basebuf, tglob, sem):
        w = lax.axis_index("s")
        base = w * CHUNK
        ones = jnp.full((16,), 1, _i32)
        zeros = jnp.full((16,), 0, _i32)
        iota = lax.iota(_i32, 16)
        shift_u = _u32(shift)

        def load_key(i):
            if first:
                f = inbuf_f[pl.ds(i, 16)]
                return _to_key(lax.bitcast_convert_type(f, _u32))
            return inbuf_u[pl.ds(i, 16)]

        def stage(blk):
            start = base + blk * BLK
            if first:
                pltpu.sync_copy(in_hbm.at[pl.ds(start, BLK)], inbuf_f)
            else:
                pltpu.sync_copy(in_hbm.at[pl.ds(start, BLK)], inbuf_u)

        @pl.loop(0, 4096, step=16)
        def _(i):
            cnt[pl.ds(i, 16)] = zeros

        @pl.loop(0, NBLK)
        def _(blk):
            stage(blk)

            @pl.loop(0, BLK, step=16)
            def _(i):
                k = load_key(i)
                d = ((k >> shift_u) & _u32(255)).astype(_i32)
                plsc.addupdate_scatter(cnt, [d * 16 + iota], ones)

        @pl.loop(0, 256, step=16)
        def _(d0):
            acc = zeros
            for l in range(16):
                acc = acc + plsc.load_gather(cnt, [(d0 + iota) * 16 + l])
            tbuf[pl.ds(d0, 16)] = acc
        pltpu.sync_copy(tbuf, tglob.at[w])
        plsc.subcore_barrier()

        pltpu.sync_copy(tglob, tall)
        for dv in range(16):
            def wbody(wi, carry):
                g, p = carry
                row = tall[wi, pl.ds(dv * 16, 16)]
                p = jnp.where(wi < w, p + row, p)
                return (g + row, p)
            g, p = lax.fori_loop(0, NW, wbody, (zeros, zeros))
            gbuf[pl.ds(dv * 16, 16)] = g
            pbuf[pl.ds(dv * 16, 16)] = p
        carry = _i32(0)
        for dv in range(16):
            g = gbuf[pl.ds(dv * 16, 16)]
            inc = plsc.cumsum(g)
            basebuf[pl.ds(dv * 16, 16)] = (inc - g + carry
                                           + pbuf[pl.ds(dv * 16, 16)])
            carry = carry + jnp.sum(g)

        @pl.loop(0, 256)
        def _(d):
            cv = cnt[pl.ds(d * 16, 16)]
            inc = plsc.cumsum(cv)
            bv = basebuf[pl.ds(d, 16)]
            off[pl.ds(d * 16, 16)] = inc - cv + bv[0]

        @pl.loop(0, NBLK)
        def _(blk):
            stage(blk)

            @pl.loop(0, BLK, step=16)
            def _(i):
                k = load_key(i)
                d = ((k >> shift_u) & _u32(255)).astype(_i32)
                addr = d * 16 + iota
                r = plsc.load_gather(off, [addr])
                plsc.store_scatter(off, [addr], r + 1)
                p = r if final else _pos_transform(r)
                row = lax.shift_right_logical(i, 7)
                col = i & 127
                posbuf[row, pl.ds(col, 16)] = p
                if first:
                    inbuf_u[pl.ds(i, 16)] = k

            @pl.loop(0, NROW, step=8)
            def _(j):
                descs = []
                for b in range(8):
                    descs.append(pltpu.async_copy(
                        inbuf_u.at[pl.ds((j + b) * 128, 128)],
                        out_hbm.at[posbuf.at[j + b]], sem))
                for dsc in descs:
                    dsc.wait()

    return body


def _radix_launch(shift, first, final):
    scratch = [
        pltpu.VMEM((BLK,), _f32),
        pltpu.VMEM((BLK,), _u32),
        pltpu.VMEM((NROW, 128), _i32),
        pltpu.VMEM((4096,), _i32),
        pltpu.VMEM((4096,), _i32),
        pltpu.VMEM((256,), _i32),
        pltpu.VMEM((NW, 256), _i32),
        pltpu.VMEM((256,), _i32),
        pltpu.VMEM((256,), _i32),
        pltpu.VMEM((272,), _i32),
        pltpu.VMEM_SHARED((NW, 256), _i32),
        pltpu.SemaphoreType.DMA,
    ]
    return pl.kernel(
        _radix_body(shift, first, final),
        out_type=jax.ShapeDtypeStruct((N,), _u32),
        mesh=_mesh(), compiler_params=_cparams(),
        scratch_types=scratch)


def _stage_dedup(a0, dbuf, w, blk):
    start = w * CHUNK + blk * BLK
    first_glob = (w == 0) & (blk == 0)

    @pl.when(first_glob)
    def _():
        pltpu.sync_copy(a0.at[pl.ds(0, BLK)], dbuf.at[pl.ds(16, BLK)])

    @pl.when(jnp.logical_not(first_glob))
    def _():
        pltpu.sync_copy(a0.at[pl.ds(start - 16, BLK + 16)], dbuf)
    return first_glob


def _count_body(a0, kout, dbuf, tbuf, sem):
    w = lax.axis_index("s")
    iota = lax.iota(_i32, 16)

    def blk_body(blk, total):
        first_glob = _stage_dedup(a0, dbuf, w, blk)

        def vbody(i, acc):
            cur = dbuf[pl.ds(16 + i * 16, 16)]
            prev = dbuf[pl.ds(15 + i * 16, 16)]
            keep = (cur != prev)
            keep = jnp.logical_or(
                keep, jnp.logical_and(first_glob & (i == 0), iota == 0))
            return acc + jnp.sum(keep.astype(_i32))
        return lax.fori_loop(0, BLK // 16, vbody, total)

    kcount = lax.fori_loop(0, NBLK, blk_body, _i32(0))

    @pl.loop(0, 128, step=16)
    def _(c):
        tbuf[pl.ds(c, 16)] = jnp.full((16,), 1, _i32) * kcount
    pltpu.sync_copy(tbuf.at[pl.ds(0, 128)], kout.at[w])


def _scatter_body(a0, kcnt, out, dbuf, valbuf, posbuf, kall, sem):
    w = lax.axis_index("s")
    iota = lax.iota(_i32, 16)
    pltpu.sync_copy(kcnt, kall)

    def bbody(wi, carry):
        bw, tot = carry
        row = kall[wi, pl.ds(0, 16)]
        return (jnp.where(wi < w, bw + row[0], bw), tot + row[0])
    base_w, ktot = lax.fori_loop(0, NW, bbody, (_i32(0), _i32(0)))

    def blk_body(blk, run):
        first_glob = _stage_dedup(a0, dbuf, w, blk)

        def vbody(i, run):
            cur = dbuf[pl.ds(16 + i * 16, 16)]
            prev = dbuf[pl.ds(15 + i * 16, 16)]
            keep = (cur != prev)
            keep = jnp.logical_or(
                keep, jnp.logical_and(first_glob & (i == 0), iota == 0))
            ki = keep.astype(_i32)
            inc = plsc.cumsum(ki) + run
            q = inc - 1
            v = _from_key(cur)
            fv = _f32(_PREF) * jnp.exp(_f32(-0.5) * v * v)
            row = lax.shift_right_logical(i, 3)
            col = (i & 7) * 16
            posbuf[row, pl.ds(col, 16)] = q
            valbuf[row, pl.ds(col, 16)] = fv
            return run + jnp.sum(ki)
        run = lax.fori_loop(0, BLK // 16, vbody, run)

        @pl.loop(0, NROW, step=8)
        def _(j):
            descs = []
            for b in range(8):
                descs.append(pltpu.async_copy(
                    valbuf.at[j + b], out.at[posbuf.at[j + b]], sem))
            for dsc in descs:
                dsc.wait()
        return run

    lax.fori_loop(0, NBLK, blk_body, base_w)

    @pl.loop(0, 128, step=16)
    def _(c):
        valbuf[0, pl.ds(c, 16)] = jnp.full((16,), _PREF, _f32)
    nrows_tail = (N - ktot + 127) // 128

    def trow(t, _):
        row = w + t * NW
        rbase = ktot + row * 128

        @pl.loop(0, 128, step=16)
        def _(c):
            posbuf[0, pl.ds(c, 16)] = jnp.minimum(rbase + c + iota, N - 1)
        pltpu.async_copy(valbuf.at[0], out.at[posbuf.at[0]], sem).wait()
        return _i32(0)

    my_rows = (nrows_tail - w + NW - 1) // NW
    lax.fori_loop(0, my_rows, trow, _i32(0))


def kernel(txv):
    xcol = txv[:, 2]
    a1 = _radix_launch(0, True, False)(xcol)
    a0 = _radix_launch(8, False, False)(a1)
    a1 = _radix_launch(16, False, False)(a0)
    a0 = _radix_launch(24, False, True)(a1)

    kcnt = pl.kernel(
        _count_body,
        out_type=jax.ShapeDtypeStruct((NW, 128), _i32),
        mesh=_mesh(), compiler_params=_cparams(),
        scratch_types=[
            pltpu.VMEM((BLK + 16,), _u32),
            pltpu.VMEM((256,), _i32),
            pltpu.SemaphoreType.DMA,
        ])(a0)

    out = pl.kernel(
        _scatter_body,
        out_type=jax.ShapeDtypeStruct((N,), _f32),
        mesh=_mesh(), compiler_params=_cparams(),
        scratch_types=[
            pltpu.VMEM((BLK + 16,), _u32),
            pltpu.VMEM((NROW, 128), _f32),
            pltpu.VMEM((NROW, 128), _i32),
            pltpu.VMEM((NW, 128), _i32),
            pltpu.SemaphoreType.DMA,
        ])(a0, kcnt)
    return out


# speedup vs baseline: 1.0043x; 1.0043x over previous
import dataclasses
from math import pi as PI, sqrt

import jax
import jax.numpy as jnp
from jax import lax
from jax.experimental import pallas as pl
from jax.experimental.pallas import tpu as pltpu
from jax.experimental.pallas import tpu_sc as plsc

N = 1048576
NW = 16
CHUNK = N // NW
BLK = 8192
NBLK = CHUNK // BLK
NROW = BLK // 128
_PREF = 1.0 / sqrt(2.0 * PI)

_i32 = jnp.int32
_u32 = jnp.uint32
_f32 = jnp.float32


def _cparams():
    cp = pltpu.CompilerParams()
    if "needs_layout_passes" in pltpu.CompilerParams.__dataclass_fields__:
        cp = dataclasses.replace(cp, needs_layout_passes=False)
    return cp


def _mesh():
    return plsc.VectorSubcoreMesh(
        core_axis_name="c", subcore_axis_name="s", num_cores=1)


def _to_key(fbits):
    top = fbits >> _u32(31)
    return jnp.where(top == _u32(1), ~fbits, fbits | _u32(0x80000000))


def _from_key(k):
    top = k >> _u32(31)
    bits = jnp.where(top == _u32(1), k & _u32(0x7FFFFFFF), ~k)
    return lax.bitcast_convert_type(bits, _f32)


def _pos_transform(r):
    w = lax.shift_right_logical(r, 16)
    rem = r & 0xFFFF
    l = lax.shift_right_logical(rem, 12)
    v = rem & 0xFFF
    return (w << 16) | (v << 4) | l


def _radix_body(shift, first, final):
    def body(in_hbm, out_hbm, inbuf_f, inbuf_u, posbuf, val2d, cnt, off,
             tbuf, tall, gbuf, pbuf, basebuf, tglob, sem):
        w = lax.axis_index("s")
        base = w * CHUNK
        ones = jnp.full((16,), 1, _i32)
        zeros = jnp.full((16,), 0, _i32)
        iota = lax.iota(_i32, 16)
        shift_u = _u32(shift)

        def load_key(i):
            if first:
                f = inbuf_f[pl.ds(i, 16)]
                return _to_key(lax.bitcast_convert_type(f, _u32))
            return inbuf_u[pl.ds(i, 16)]

        def stage(blk):
            start = base + blk * BLK
            if first:
                pltpu.sync_copy(in_hbm.at[pl.ds(start, BLK)], inbuf_f)
            else:
                pltpu.sync_copy(in_hbm.at[pl.ds(start, BLK)], inbuf_u)

        @pl.loop(0, 4096, step=16)
        def _(i):
            cnt[pl.ds(i, 16)] = zeros

        @pl.loop(0, NBLK)
        def _(blk):
            stage(blk)

            @pl.loop(0, BLK, step=16)
            def _(i):
                k = load_key(i)
                d = ((k >> shift_u) & _u32(255)).astype(_i32)
                plsc.addupdate_scatter(cnt, [d * 16 + iota], ones)

        @pl.loop(0, 256, step=16)
        def _(d0):
            acc = zeros
            for l in range(16):
                acc = acc + plsc.load_gather(cnt, [(d0 + iota) * 16 + l])
            tbuf[pl.ds(d0, 16)] = acc
        pltpu.sync_copy(tbuf, tglob.at[w])
        plsc.subcore_barrier()

        pltpu.sync_copy(tglob, tall)
        for dv in range(16):
            def wbody(wi, carry):
                g, p = carry
                row = tall[wi, pl.ds(dv * 16, 16)]
                p = jnp.where(wi < w, p + row, p)
                return (g + row, p)
            g, p = lax.fori_loop(0, NW, wbody, (zeros, zeros))
            gbuf[pl.ds(dv * 16, 16)] = g
            pbuf[pl.ds(dv * 16, 16)] = p
        carry = _i32(0)
        for dv in range(16):
            g = gbuf[pl.ds(dv * 16, 16)]
            inc = plsc.cumsum(g)
            basebuf[pl.ds(dv * 16, 16)] = (inc - g + carry
                                           + pbuf[pl.ds(dv * 16, 16)])
            carry = carry + jnp.sum(g)

        @pl.loop(0, 256)
        def _(d):
            cv = cnt[pl.ds(d * 16, 16)]
            inc = plsc.cumsum(cv)
            bv = basebuf[pl.ds(d, 16)]
            off[pl.ds(d * 16, 16)] = inc - cv + bv[0]

        @pl.loop(0, NBLK)
        def _(blk):
            stage(blk)

            @pl.loop(0, BLK, step=16)
            def _(i):
                k = load_key(i)
                d = ((k >> shift_u) & _u32(255)).astype(_i32)
                addr = d * 16 + iota
                r = plsc.load_gather(off, [addr])
                plsc.store_scatter(off, [addr], r + 1)
                p = r if final else _pos_transform(r)
                posbuf[pl.ds(i, 16)] = p
                val2d[pl.ds(i, 16)] = k

            pltpu.async_copy(val2d, out_hbm.at[posbuf], sem).wait()

    return body


def _radix_launch(shift, first, final):
    scratch = [
        pltpu.VMEM((BLK,), _f32),
        pltpu.VMEM((BLK,), _u32),
        pltpu.VMEM((BLK,), _i32),
        pltpu.VMEM((BLK,), _u32),
        pltpu.VMEM((4096,), _i32),
        pltpu.VMEM((4096,), _i32),
        pltpu.VMEM((256,), _i32),
        pltpu.VMEM((NW, 256), _i32),
        pltpu.VMEM((256,), _i32),
        pltpu.VMEM((256,), _i32),
        pltpu.VMEM((272,), _i32),
        pltpu.VMEM_SHARED((NW, 256), _i32),
        pltpu.SemaphoreType.DMA,
    ]
    return pl.kernel(
        _radix_body(shift, first, final),
        out_type=jax.ShapeDtypeStruct((N,), _u32),
        mesh=_mesh(), compiler_params=_cparams(),
        scratch_types=scratch)


def _stage_dedup(a0, dbuf, w, blk):
    start = w * CHUNK + blk * BLK
    first_glob = (w == 0) & (blk == 0)

    @pl.when(first_glob)
    def _():
        pltpu.sync_copy(a0.at[pl.ds(0, BLK)], dbuf.at[pl.ds(16, BLK)])

    @pl.when(jnp.logical_not(first_glob))
    def _():
        pltpu.sync_copy(a0.at[pl.ds(start - 16, BLK + 16)], dbuf)
    return first_glob


def _count_body(a0, kout, dbuf, tbuf, sem):
    w = lax.axis_index("s")
    iota = lax.iota(_i32, 16)

    def blk_body(blk, total):
        first_glob = _stage_dedup(a0, dbuf, w, blk)

        def vbody(i, acc):
            cur = dbuf[pl.ds(16 + i * 16, 16)]
            prev = dbuf[pl.ds(15 + i * 16, 16)]
            keep = (cur != prev)
            keep = jnp.logical_or(
                keep, jnp.logical_and(first_glob & (i == 0), iota == 0))
            return acc + jnp.sum(keep.astype(_i32))
        return lax.fori_loop(0, BLK // 16, vbody, total)

    kcount = lax.fori_loop(0, NBLK, blk_body, _i32(0))

    @pl.loop(0, 128, step=16)
    def _(c):
        tbuf[pl.ds(c, 16)] = jnp.full((16,), 1, _i32) * kcount
    pltpu.sync_copy(tbuf.at[pl.ds(0, 128)], kout.at[w])


def _scatter_body(a0, kcnt, out, dbuf, valbuf, posbuf, kall, sem):
    w = lax.axis_index("s")
    iota = lax.iota(_i32, 16)
    pltpu.sync_copy(kcnt, kall)

    def bbody(wi, carry):
        bw, tot = carry
        row = kall[wi, pl.ds(0, 16)]
        return (jnp.where(wi < w, bw + row[0], bw), tot + row[0])
    base_w, ktot = lax.fori_loop(0, NW, bbody, (_i32(0), _i32(0)))

    def blk_body(blk, run):
        first_glob = _stage_dedup(a0, dbuf, w, blk)

        def vbody(i, run):
            cur = dbuf[pl.ds(16 + i * 16, 16)]
            prev = dbuf[pl.ds(15 + i * 16, 16)]
            keep = (cur != prev)
            keep = jnp.logical_or(
                keep, jnp.logical_and(first_glob & (i == 0), iota == 0))
            ki = keep.astype(_i32)
            inc = plsc.cumsum(ki) + run
            q = inc - 1
            v = _from_key(cur)
            fv = _f32(_PREF) * jnp.exp(_f32(-0.5) * v * v)
            posbuf[pl.ds(i * 16, 16)] = q
            valbuf[pl.ds(i * 16, 16)] = fv
            return run + jnp.sum(ki)
        run = lax.fori_loop(0, BLK // 16, vbody, run)
        pltpu.async_copy(valbuf, out.at[posbuf], sem).wait()
        return run

    lax.fori_loop(0, NBLK, blk_body, base_w)

    @pl.loop(0, 128, step=16)
    def _(c):
        valbuf[pl.ds(c, 16)] = jnp.full((16,), _PREF, _f32)
    nrows_tail = (N - ktot + 127) // 128

    def trow(t, _):
        row = w + t * NW
        rbase = ktot + row * 128

        @pl.loop(0, 128, step=16)
        def _(c):
            posbuf[pl.ds(c, 16)] = jnp.minimum(rbase + c + iota, N - 1)
        pltpu.async_copy(valbuf.at[pl.ds(0, 128)],
                         out.at[posbuf.at[pl.ds(0, 128)]], sem).wait()
        return _i32(0)

    my_rows = (nrows_tail - w + NW - 1) // NW
    lax.fori_loop(0, my_rows, trow, _i32(0))


def kernel(txv):
    xcol = txv[:, 2]
    a1 = _radix_launch(0, True, False)(xcol)
    a0 = _radix_launch(8, False, False)(a1)
    a1 = _radix_launch(16, False, False)(a0)
    a0 = _radix_launch(24, False, True)(a1)

    kcnt = pl.kernel(
        _count_body,
        out_type=jax.ShapeDtypeStruct((NW, 128), _i32),
        mesh=_mesh(), compiler_params=_cparams(),
        scratch_types=[
            pltpu.VMEM((BLK + 16,), _u32),
            pltpu.VMEM((256,), _i32),
            pltpu.SemaphoreType.DMA,
        ])(a0)

    out = pl.kernel(
        _scatter_body,
        out_type=jax.ShapeDtypeStruct((N,), _f32),
        mesh=_mesh(), compiler_params=_cparams(),
        scratch_types=[
            pltpu.VMEM((BLK + 16,), _u32),
            pltpu.VMEM((BLK,), _f32),
            pltpu.VMEM((BLK,), _i32),
            pltpu.VMEM((NW, 128), _i32),
            pltpu.SemaphoreType.DMA,
        ])(a0, kcnt)
    return out
